# Initial kernel scaffold; baseline (speedup 1.0000x reference)
import jax
import jax.numpy as jnp
from jax.experimental import pallas as pl


def kernel(x, edge_index, params):
    raise NotImplementedError("write your pallas kernel here")


# speedup vs baseline: 7.9771x; 7.9771x over previous
import functools
import math

import jax
import jax.numpy as jnp
from jax import lax
from jax.experimental import pallas as pl
from jax.experimental.pallas import tpu as pltpu
from jax.experimental.pallas import tpu_sc as plsc

N = 10000
NPAD = 10240
E = 320000
NW = 32
B = 80
ROW_BLK = 1000
LANES = 16

_F32 = jnp.float32


def _mesh():
    return plsc.VectorSubcoreMesh(
        core_axis_name="c", subcore_axis_name="s", num_cores=2, num_subcores=16
    )


def _edge_block_ex(qr, kr, tb, ex, nchunk):
    col_iota = lax.iota(jnp.int32, LANES) * LANES

    def eblock(eb, _):
        base_e = eb * LANES

        def one_edge(j, _):
            e = base_e + j
            acc = qr[e, pl.ds(0, LANES)] * kr[e, pl.ds(0, LANES)]
            for c in range(1, nchunk):
                acc = acc + qr[e, pl.ds(c * LANES, LANES)] * kr[e, pl.ds(c * LANES, LANES)]
            plsc.store_scatter(tb, [col_iota + j], acc)
            return 0

        lax.fori_loop(0, LANES, one_edge, 0)
        alpha = tb[pl.ds(0, LANES)]
        for r in range(1, LANES):
            alpha = alpha + tb[pl.ds(r * LANES, LANES)]
        ex[pl.ds(base_e, LANES)] = jnp.exp(alpha)
        return 0

    lax.fori_loop(0, B // LANES, eblock, 0)


def _zero_vmem(ref, rows, width):
    z = jnp.zeros((LANES,), _F32)

    def body(r, _):
        for c in range(width // LANES):
            ref[r, pl.ds(c * LANES, LANES)] = z
        return 0

    lax.fori_loop(0, rows, body, 0)


def _zero_spmem_slice(acc_sp, zrow, sid, zr, width):
    rows_per_tile = NPAD // 16
    base = sid * rows_per_tile
    for j in range(rows_per_tile // zr):
        pltpu.sync_copy(zrow, acc_sp.at[pl.ds(base + j * zr, zr)])


def _sc_fused(d, dpad, w):
    ew = E // NW
    nb = ew // B
    zr = 128
    nchunk = dpad // LANES

    @functools.partial(
        pl.kernel,
        out_type=jax.ShapeDtypeStruct((2, NPAD, w), _F32),
        mesh=_mesh(),
        compiler_params=pltpu.CompilerParams(needs_layout_passes=False, use_tc_tiling_on_sc=False),
        scratch_types=[
            pltpu.VMEM((B,), jnp.int32),
            pltpu.VMEM((B,), jnp.int32),
            pltpu.VMEM((B, dpad), _F32),
            pltpu.VMEM((B, dpad), _F32),
            pltpu.VMEM((B, dpad), _F32),
            pltpu.VMEM((B, w), _F32),
            pltpu.VMEM((B,), _F32),
            pltpu.VMEM((LANES * LANES,), _F32),
            pltpu.VMEM((zr, w), _F32),
            pltpu.VMEM_SHARED((NPAD, w), _F32),
            pltpu.SemaphoreType.DMA,
        ],
    )
    def kern(q_hbm, k_hbm, v_hbm, src_hbm, dst_hbm, out_hbm,
             src_i, dst_i, qr, kr, vr, sc, ex, tb, zrow, acc_sp, sem):
        cid = lax.axis_index("c")
        sid = lax.axis_index("s")
        wid = cid * 16 + sid

        _zero_vmem(zrow, zr, w)
        _zero_spmem_slice(acc_sp, zrow, sid, zr, w)
        plsc.subcore_barrier()

        def batch(b, _):
            base = wid * ew + b * B
            pltpu.sync_copy(src_hbm.at[pl.ds(base, B)], src_i)
            pltpu.sync_copy(dst_hbm.at[pl.ds(base, B)], dst_i)
            cq = pltpu.async_copy(q_hbm.at[dst_i], qr, sem)
            ck = pltpu.async_copy(k_hbm.at[src_i], kr, sem)
            cv = pltpu.async_copy(v_hbm.at[src_i], vr, sem)
            cq.wait()
            ck.wait()
            cv.wait()

            _edge_block_ex(qr, kr, tb, ex, nchunk)

            def escale(eb, _):
                base_e = eb * LANES
                exvec = ex[pl.ds(base_e, LANES)]
                for j in range(LANES):
                    e = base_e + j
                    s = jnp.full((LANES,), exvec[j], _F32)
                    for c in range(nchunk):
                        sc[e, pl.ds(c * LANES, LANES)] = vr[e, pl.ds(c * LANES, LANES)] * s
                    sc[e, pl.ds(dpad, LANES)] = s
                return 0

            lax.fori_loop(0, B // LANES, escale, 0)
            pltpu.sync_copy(sc, acc_sp.at[dst_i], add=True)
            return 0

        lax.fori_loop(0, nb, batch, 0)
        plsc.subcore_barrier()
        rows_per_tile = NPAD // 16
        pltpu.sync_copy(
            acc_sp.at[pl.ds(sid * rows_per_tile, rows_per_tile)],
            out_hbm.at[cid, pl.ds(sid * rows_per_tile, rows_per_tile)],
        )

    return kern


def _sc_alpha_l1(dpad):
    ew = E // NW
    nb = ew // B
    zr = 128
    w = LANES
    nchunk = dpad // LANES

    @functools.partial(
        pl.kernel,
        out_type=(
            jax.ShapeDtypeStruct((E,), _F32),
            jax.ShapeDtypeStruct((2, NPAD, w), _F32),
        ),
        mesh=_mesh(),
        compiler_params=pltpu.CompilerParams(needs_layout_passes=False, use_tc_tiling_on_sc=False),
        scratch_types=[
            pltpu.VMEM((B,), jnp.int32),
            pltpu.VMEM((B,), jnp.int32),
            pltpu.VMEM((B, dpad), _F32),
            pltpu.VMEM((B, dpad), _F32),
            pltpu.VMEM((B, w), _F32),
            pltpu.VMEM((B,), _F32),
            pltpu.VMEM((LANES * LANES,), _F32),
            pltpu.VMEM((zr, w), _F32),
            pltpu.VMEM_SHARED((NPAD, w), _F32),
            pltpu.SemaphoreType.DMA,
        ],
    )
    def kern(q_hbm, k_hbm, src_hbm, dst_hbm, ex_hbm, den_hbm,
             src_i, dst_i, qr, kr, sc, ex, tb, zrow, acc_sp, sem):
        cid = lax.axis_index("c")
        sid = lax.axis_index("s")
        wid = cid * 16 + sid

        _zero_vmem(zrow, zr, w)
        _zero_spmem_slice(acc_sp, zrow, sid, zr, w)
        plsc.subcore_barrier()

        def batch(b, _):
            base = wid * ew + b * B
            pltpu.sync_copy(src_hbm.at[pl.ds(base, B)], src_i)
            pltpu.sync_copy(dst_hbm.at[pl.ds(base, B)], dst_i)
            cq = pltpu.async_copy(q_hbm.at[dst_i], qr, sem)
            ck = pltpu.async_copy(k_hbm.at[src_i], kr, sem)
            cq.wait()
            ck.wait()

            _edge_block_ex(qr, kr, tb, ex, nchunk)

            def espread(eb, _):
                base_e = eb * LANES
                exvec = ex[pl.ds(base_e, LANES)]
                for j in range(LANES):
                    sc[base_e + j, pl.ds(0, LANES)] = jnp.full((LANES,), exvec[j], _F32)
                return 0

            lax.fori_loop(0, B // LANES, espread, 0)
            pltpu.sync_copy(ex, ex_hbm.at[pl.ds(base, B)])
            pltpu.sync_copy(sc, acc_sp.at[dst_i], add=True)
            return 0

        lax.fori_loop(0, nb, batch, 0)
        plsc.subcore_barrier()
        rows_per_tile = NPAD // 16
        pltpu.sync_copy(
            acc_sp.at[pl.ds(sid * rows_per_tile, rows_per_tile)],
            den_hbm.at[cid, pl.ds(sid * rows_per_tile, rows_per_tile)],
        )

    return kern


def _sc_msg_l1():
    wc = 128
    ew = E // 16
    nb = ew // B
    zr = 128

    @functools.partial(
        pl.kernel,
        out_type=jax.ShapeDtypeStruct((4, NPAD, wc), _F32),
        mesh=_mesh(),
        compiler_params=pltpu.CompilerParams(needs_layout_passes=False, use_tc_tiling_on_sc=False),
        scratch_types=[
            pltpu.VMEM((B,), jnp.int32),
            pltpu.VMEM((B,), jnp.int32),
            pltpu.VMEM((B, wc), _F32),
            pltpu.VMEM((B, wc), _F32),
            pltpu.VMEM((B,), _F32),
            pltpu.VMEM((zr, wc), _F32),
            pltpu.VMEM_SHARED((NPAD, wc), _F32),
            pltpu.SemaphoreType.DMA,
        ],
    )
    def kern(v0_hbm, v1_hbm, v2_hbm, v3_hbm, src_hbm, dst_hbm, ex_hbm, msg_hbm,
             src_i, dst_i, vr, sc, ex, zrow, acc_sp, sem):
        cid = lax.axis_index("c")
        sid = lax.axis_index("s")
        rows_per_tile = NPAD // 16

        _zero_vmem(zrow, zr, wc)
        tables = (v0_hbm, v1_hbm, v2_hbm, v3_hbm)
        for cc in range(4):

            @pl.when(cid == cc // 2)
            def _():
                _zero_spmem_slice(acc_sp, zrow, sid, zr, wc)
                plsc.subcore_barrier()

                def batch(b, _):
                    base = sid * ew + b * B
                    pltpu.sync_copy(src_hbm.at[pl.ds(base, B)], src_i)
                    pltpu.sync_copy(dst_hbm.at[pl.ds(base, B)], dst_i)
                    pltpu.sync_copy(ex_hbm.at[pl.ds(base, B)], ex)
                    cv = pltpu.async_copy(tables[cc].at[src_i], vr, sem)
                    cv.wait()

                    def escale(eb, _):
                        base_e = eb * LANES
                        exvec = ex[pl.ds(base_e, LANES)]
                        for j in range(LANES):
                            e = base_e + j
                            s = jnp.full((LANES,), exvec[j], _F32)
                            for c in range(wc // LANES):
                                sc[e, pl.ds(c * LANES, LANES)] = vr[e, pl.ds(c * LANES, LANES)] * s
                        return 0

                    lax.fori_loop(0, B // LANES, escale, 0)
                    pltpu.sync_copy(sc, acc_sp.at[dst_i], add=True)
                    return 0

                lax.fori_loop(0, nb, batch, 0)
                plsc.subcore_barrier()
                pltpu.sync_copy(
                    acc_sp.at[pl.ds(sid * rows_per_tile, rows_per_tile)],
                    msg_hbm.at[cc, pl.ds(sid * rows_per_tile, rows_per_tile)],
                )
                plsc.subcore_barrier()

    return kern


def _matmul(h, wall_ref, ball_ref):
    y = lax.dot_general(
        h, wall_ref[...], (((1,), (0,)), ((), ())),
        precision=lax.Precision.HIGHEST, preferred_element_type=_F32,
    )
    return y + ball_ref[...]


def _split_stores(y, outs, widths):
    off = 0
    for ref, wdt in zip(outs, widths):
        ref[...] = y[:, off:off + wdt]
        off += wdt


def _full_spec(shape):
    nd = len(shape)
    return pl.BlockSpec(shape, lambda i, _n=nd: (0,) * _n)


def _row_spec(shape, axis):
    blk = tuple(ROW_BLK if a == axis else s for a, s in enumerate(shape))

    def imap(i, _axis=axis, _nd=len(shape)):
        return tuple(i if a == _axis else 0 for a in range(_nd))

    return pl.BlockSpec(blk, imap)


def _tc_stage1(qkpad, vc, d, cols):
    def body(x_ref, wall_ref, ball_ref, q_ref, k_ref, v0, v1, v2, v3, r_ref):
        y = _matmul(x_ref[...], wall_ref, ball_ref)
        _split_stores(y, (q_ref, k_ref, v0, v1, v2, v3, r_ref),
                      (qkpad, qkpad, vc, vc, vc, vc, d))

    grid = (N // ROW_BLK,)
    return pl.pallas_call(
        body,
        grid=grid,
        in_specs=[
            _row_spec((N, 128), 0),
            _full_spec((128, cols)),
            _full_spec((1, cols)),
        ],
        out_specs=[
            _row_spec((N, qkpad), 0), _row_spec((N, qkpad), 0),
            _row_spec((N, vc), 0), _row_spec((N, vc), 0),
            _row_spec((N, vc), 0), _row_spec((N, vc), 0),
            _row_spec((N, d), 0),
        ],
        out_shape=[
            jax.ShapeDtypeStruct((N, qkpad), _F32),
            jax.ShapeDtypeStruct((N, qkpad), _F32),
            jax.ShapeDtypeStruct((N, vc), _F32),
            jax.ShapeDtypeStruct((N, vc), _F32),
            jax.ShapeDtypeStruct((N, vc), _F32),
            jax.ShapeDtypeStruct((N, vc), _F32),
            jax.ShapeDtypeStruct((N, d), _F32),
        ],
    )


def _epi(msg, den, r, act):
    h = msg / (den + 1e-16) + r
    if act == "elu":
        return jnp.where(h > 0, h, jnp.exp(jnp.minimum(h, 0.0)) - 1.0)
    return jnp.maximum(h, 0.0)


def _tc_stage2(din, act, qkpad, vpad, d, cols):
    def body(m_ref, den_ref, r_ref, wall_ref, ball_ref,
             q_ref, k_ref, v_ref, rr_ref):
        msg = jnp.concatenate([m_ref[j] for j in range(4)], axis=-1)
        den = den_ref[0][:, 0:1] + den_ref[1][:, 0:1]
        h = _epi(msg, den, r_ref[...], act)
        y = _matmul(h, wall_ref, ball_ref)
        _split_stores(y, (q_ref, k_ref, v_ref, rr_ref), (qkpad, qkpad, vpad, d))

    grid = (N // ROW_BLK,)
    return pl.pallas_call(
        body,
        grid=grid,
        in_specs=[
            _row_spec((4, NPAD, 128), 1),
            _row_spec((2, NPAD, LANES), 1),
            _row_spec((N, din), 0),
            _full_spec((din, cols)),
            _full_spec((1, cols)),
        ],
        out_specs=[
            _row_spec((N, qkpad), 0), _row_spec((N, qkpad), 0),
            _row_spec((N, vpad), 0), _row_spec((N, d), 0),
        ],
        out_shape=[
            jax.ShapeDtypeStruct((N, qkpad), _F32),
            jax.ShapeDtypeStruct((N, qkpad), _F32),
            jax.ShapeDtypeStruct((N, vpad), _F32),
            jax.ShapeDtypeStruct((N, d), _F32),
        ],
    )


def _tc_stage_mid(din, dpad_prev, w_prev, act, qkpad, vpad, d, cols):
    def body(md_ref, r_ref, wall_ref, ball_ref, q_ref, k_ref, v_ref, rr_ref):
        msg = md_ref[0][:, :din] + md_ref[1][:, :din]
        den = md_ref[0][:, dpad_prev:dpad_prev + 1] + md_ref[1][:, dpad_prev:dpad_prev + 1]
        h = _epi(msg, den, r_ref[...], act)
        y = _matmul(h, wall_ref, ball_ref)
        _split_stores(y, (q_ref, k_ref, v_ref, rr_ref), (qkpad, qkpad, vpad, d))

    grid = (N // ROW_BLK,)
    return pl.pallas_call(
        body,
        grid=grid,
        in_specs=[
            _row_spec((2, NPAD, w_prev), 1),
            _row_spec((N, din), 0),
            _full_spec((din, cols)),
            _full_spec((1, cols)),
        ],
        out_specs=[
            _row_spec((N, qkpad), 0), _row_spec((N, qkpad), 0),
            _row_spec((N, vpad), 0), _row_spec((N, d), 0),
        ],
        out_shape=[
            jax.ShapeDtypeStruct((N, qkpad), _F32),
            jax.ShapeDtypeStruct((N, qkpad), _F32),
            jax.ShapeDtypeStruct((N, vpad), _F32),
            jax.ShapeDtypeStruct((N, d), _F32),
        ],
    )


def _tc_final(din, dpad_prev, w_prev, act):
    def body(md_ref, r_ref, out_ref):
        msg = md_ref[0][:, :din] + md_ref[1][:, :din]
        den = md_ref[0][:, dpad_prev:dpad_prev + 1] + md_ref[1][:, dpad_prev:dpad_prev + 1]
        out_ref[...] = _epi(msg, den, r_ref[...], act)

    grid = (N // ROW_BLK,)
    return pl.pallas_call(
        body,
        grid=grid,
        in_specs=[
            _row_spec((2, NPAD, w_prev), 1),
            _row_spec((N, din), 0),
        ],
        out_specs=_row_spec((N, din), 0),
        out_shape=jax.ShapeDtypeStruct((N, din), _F32),
    )


def _pack_params(p, d):
    wq, bq, wk, bk, wv, bv, ws, bs = p
    din = wq.shape[0]
    qkpad = max(LANES, d)
    s = 1.0 / math.sqrt(float(d))

    def padc(m, b, width, scale=1.0):
        mp = jnp.zeros((din, width), _F32).at[:, :d].set(m * scale)
        bp = jnp.zeros((width,), _F32).at[:d].set(b * scale)
        return mp, bp

    wqp, bqp = padc(wq, bq, qkpad, s)
    wkp, bkp = padc(wk, bk, qkpad)
    wvp, bvp = padc(wv, bv, qkpad)
    wall = jnp.concatenate([wqp, wkp, wvp, ws], axis=1)
    ball = jnp.concatenate([bqp, bkp, bvp, bs])[None, :]
    return wall, ball, qkpad


def kernel(x, edge_index, params):
    src = edge_index[0]
    dst = edge_index[1]

    dims = [p[0].shape[1] for p in params]
    acts = ["elu", "elu", "relu", "relu", "relu"]
    qk = [max(LANES, d) for d in dims]
    vpad = [max(LANES, d) for d in dims]
    wacc = [v + LANES for v in vpad]

    wall1, ball1, _ = _pack_params(params[0], dims[0])
    q1, k1, v10, v11, v12, v13, r1 = _tc_stage1(qk[0], 128, dims[0], wall1.shape[1])(
        x, wall1, ball1)
    ex1, den1 = _sc_alpha_l1(qk[0])(q1, k1, src, dst)
    msg1 = _sc_msg_l1()(v10, v11, v12, v13, src, dst, ex1)

    wall2, ball2, _ = _pack_params(params[1], dims[1])
    q2, k2, v2, r2 = _tc_stage2(dims[0], acts[0], qk[1], vpad[1], dims[1],
                                wall2.shape[1])(msg1, den1, r1, wall2, ball2)
    md2 = _sc_fused(dims[1], vpad[1], wacc[1])(q2, k2, v2, src, dst)

    md_prev, r_prev = md2, r2
    for li in (2, 3, 4):
        wall, ball, _ = _pack_params(params[li], dims[li])
        stage = _tc_stage_mid(dims[li - 1], vpad[li - 1], wacc[li - 1],
                              acts[li - 1], qk[li], vpad[li], dims[li],
                              wall.shape[1])
        qn, kn, vn, rn = stage(md_prev, r_prev, wall, ball)
        md_prev = _sc_fused(dims[li], vpad[li], wacc[li])(qn, kn, vn, src, dst)
        r_prev = rn

    return _tc_final(dims[4], vpad[4], wacc[4], acts[4])(md_prev, r_prev)
